# Initial kernel scaffold; baseline (speedup 1.0000x reference)
import jax
import jax.numpy as jnp
from jax.experimental import pallas as pl


def kernel(x, calib, emb, W_down, b_down, W_up, b_up):
    raise NotImplementedError("write your pallas kernel here")


# speedup vs baseline: 2.0146x; 2.0146x over previous
import functools

import jax
import jax.numpy as jnp
from jax.experimental import pallas as pl
from jax.experimental.pallas import tpu as pltpu
from jax.experimental.pallas import tpu_sc as plsc

_B, _L, _C = 32, 450, 768
_MID = 256
_K = 8912
_KP = 9216
_N = _B * _L
_TN = 1800
_TK = 512
_NT = _N // _TN
_NK = _KP // _TK
_TK0 = 1024
_INV_TEMP = -1.0 / 0.07
_EPS = 1e-12

_INTERPRET = False


def _prep_body(emb_ref, en_ref, enT_ref):
    e = emb_ref[...]
    n = jnp.sqrt(jnp.sum(e * e, axis=1, keepdims=True))
    en = e / jnp.maximum(n, _EPS)
    en_ref[...] = en
    enT_ref[...] = en.T


def _up_body(zq_ref, wup_ref, bup_ref, out_ref):
    out_ref[...] = jnp.dot(zq_ref[...], wup_ref[...],
                           preferred_element_type=jnp.float32) + bup_ref[...]


def _flash_body(x_ref, wd_ref, bd_ref, enT_ref, dmin_ref, idx_ref, se_ref,
                zn_ref, rmin_ref, ridx_ref, rsum_ref):
    k = pl.program_id(1)

    @pl.when(k == 0)
    def _():
        z = jnp.dot(x_ref[...], wd_ref[...],
                    preferred_element_type=jnp.float32) + bd_ref[...]
        n = jnp.sqrt(jnp.sum(z * z, axis=1, keepdims=True))
        zn_ref[...] = z / jnp.maximum(n, _EPS)
        rmin_ref[...] = jnp.full((_TN, 1), 3e38, jnp.float32)
        ridx_ref[...] = jnp.full((_TN, 1), 2**30, jnp.int32)
        rsum_ref[...] = jnp.zeros((_TN, 1), jnp.float32)

    en_k = enT_ref[:, pl.ds(k * _TK, _TK)]
    s = jnp.dot(zn_ref[...], en_k, preferred_element_type=jnp.float32)
    d = -2.0 * s + 2.0
    lane = jax.lax.broadcasted_iota(jnp.int32, (1, _TK), 1)
    tail_bias = jnp.where((lane + k * _TK) >= _K, jnp.float32(1e30),
                          jnp.float32(0.0))
    d = d + tail_bias
    tmin = jnp.min(d, axis=1, keepdims=True)
    col = jax.lax.broadcasted_iota(jnp.int32, (_TN, _TK), 1) + k * _TK
    tidx = jnp.min(jnp.where(d == tmin, col, jnp.int32(2**30)),
                   axis=1, keepdims=True)
    e = jnp.exp(d * jnp.float32(_INV_TEMP))
    tsum = jnp.sum(e, axis=1, keepdims=True)

    better = tmin < rmin_ref[...]
    ridx_ref[...] = jnp.where(better, tidx, ridx_ref[...])
    rmin_ref[...] = jnp.where(better, tmin, rmin_ref[...])
    rsum_ref[...] = rsum_ref[...] + tsum

    @pl.when(k == _NK - 1)
    def _():
        dmin_ref[...] = rmin_ref[...]
        idx_ref[...] = ridx_ref[...]
        se_ref[...] = rsum_ref[...]


def _scalars_body(dmin_ref, se_ref, idx_ref, vq_ref, use_ref, top1_ref):
    dm = dmin_ref[...]
    se = se_ref[...]
    vq_ref[...] = (jnp.sum(dm) / jnp.float32(_N * _MID)).reshape(1, 1)
    t1 = jnp.exp(dm * jnp.float32(_INV_TEMP)) / se
    top1_ref[...] = (jnp.sum(t1) / jnp.float32(_N)).reshape(1, 1)

    c_chunk, t_chunk = 1024, 1800

    def code_tile(j, count):
        codes = jax.lax.broadcasted_iota(jnp.int32, (1, c_chunk), 1) \
            + j * c_chunk

        def tok_tile(i, acc):
            chunk = idx_ref[pl.ds(i * t_chunk, t_chunk), :]
            eq = (chunk == codes).astype(jnp.float32)
            return jnp.maximum(acc, jnp.max(eq, axis=0, keepdims=True))

        pres = jax.lax.fori_loop(
            0, _N // t_chunk, tok_tile,
            jnp.zeros((1, c_chunk), jnp.float32))
        return count + jnp.sum(pres)

    count = jax.lax.fori_loop(0, _KP // c_chunk, code_tile, jnp.float32(0.0))
    use_ref[...] = (count / jnp.float32(_K)).reshape(1, 1)


_NG = 14464


def _sc_gather(en, idx2):
    mesh = plsc.VectorSubcoreMesh(core_axis_name="core",
                                  subcore_axis_name="subcore")
    win = 128

    @functools.partial(
        pl.kernel,
        out_type=jax.ShapeDtypeStruct((_NG, _MID), jnp.float32),
        mesh=mesh)
    def gk(en_hbm, i_hbm, o_hbm):
        def body(i_vmem, o_vmem):
            pltpu.sync_copy(en_hbm.at[i_vmem.at[0]], o_vmem)

        pltpu.emit_pipeline(
            body,
            grid=(_NG // win,),
            in_specs=[pl.BlockSpec((1, win), lambda i: (0, i))],
            out_specs=[pl.BlockSpec((win, _MID), lambda i: (i, 0))],
            core_axis_name=("core", "subcore"),
            dimension_semantics=(pltpu.PARALLEL,),
        )(i_hbm, o_hbm)

    return gk(en, idx2)


def kernel(x, calib, emb, W_down, b_down, W_up, b_up):
    del calib
    xf = x.reshape(_N, _C)
    embP = jnp.pad(emb, ((0, _KP - _K), (0, 0)))

    en, enT = pl.pallas_call(
        _prep_body,
        grid=(_KP // _TK0,),
        in_specs=[
            pl.BlockSpec((_TK0, _MID), lambda k: (k, 0)),
        ],
        out_specs=[
            pl.BlockSpec((_TK0, _MID), lambda k: (k, 0)),
            pl.BlockSpec((_MID, _TK0), lambda k: (0, k)),
        ],
        out_shape=[
            jax.ShapeDtypeStruct((_KP, _MID), jnp.float32),
            jax.ShapeDtypeStruct((_MID, _KP), jnp.float32),
        ],
        interpret=_INTERPRET,
    )(embP)

    dmin, idx, se = pl.pallas_call(
        _flash_body,
        grid=(_NT, _NK),
        in_specs=[
            pl.BlockSpec((_TN, _C), lambda t, k: (t, 0)),
            pl.BlockSpec((_C, _MID), lambda t, k: (0, 0)),
            pl.BlockSpec((1, _MID), lambda t, k: (0, 0)),
            pl.BlockSpec((_MID, _KP), lambda t, k: (0, 0)),
        ],
        out_specs=[
            pl.BlockSpec((_TN, 1), lambda t, k: (t, 0)),
            pl.BlockSpec((_TN, 1), lambda t, k: (t, 0)),
            pl.BlockSpec((_TN, 1), lambda t, k: (t, 0)),
        ],
        out_shape=[
            jax.ShapeDtypeStruct((_N, 1), jnp.float32),
            jax.ShapeDtypeStruct((_N, 1), jnp.int32),
            jax.ShapeDtypeStruct((_N, 1), jnp.float32),
        ],
        scratch_shapes=[
            pltpu.VMEM((_TN, _MID), jnp.float32),
            pltpu.VMEM((_TN, 1), jnp.float32),
            pltpu.VMEM((_TN, 1), jnp.int32),
            pltpu.VMEM((_TN, 1), jnp.float32),
        ],
        interpret=_INTERPRET,
    )(xf, W_down, b_down.reshape(1, _MID), enT)

    vq_s, use_s, top1_s = pl.pallas_call(
        _scalars_body,
        in_specs=[
            pl.BlockSpec((_N, 1), lambda: (0, 0)),
            pl.BlockSpec((_N, 1), lambda: (0, 0)),
            pl.BlockSpec((_N, 1), lambda: (0, 0)),
        ],
        out_specs=[
            pl.BlockSpec((1, 1), lambda: (0, 0)),
            pl.BlockSpec((1, 1), lambda: (0, 0)),
            pl.BlockSpec((1, 1), lambda: (0, 0)),
        ],
        out_shape=[
            jax.ShapeDtypeStruct((1, 1), jnp.float32),
            jax.ShapeDtypeStruct((1, 1), jnp.float32),
            jax.ShapeDtypeStruct((1, 1), jnp.float32),
        ],
        interpret=_INTERPRET,
    )(dmin, se, idx)

    if _INTERPRET:
        zq = jnp.take(en, idx[:, 0], axis=0)
    else:
        idxp = jnp.pad(idx.reshape(1, _N), ((0, 0), (0, _NG - _N)))
        zq = _sc_gather(en, idxp)[:_N]

    out2d = pl.pallas_call(
        _up_body,
        grid=(_NT,),
        in_specs=[
            pl.BlockSpec((_TN, _MID), lambda t: (t, 0)),
            pl.BlockSpec((_MID, _C), lambda t: (0, 0)),
            pl.BlockSpec((1, _C), lambda t: (0, 0)),
        ],
        out_specs=pl.BlockSpec((_TN, _C), lambda t: (t, 0)),
        out_shape=jax.ShapeDtypeStruct((_N, _C), jnp.float32),
        interpret=_INTERPRET,
    )(zq, W_up, b_up.reshape(1, _C))

    out = out2d.reshape(_B, _L, _C)
    vq = vq_s[0, 0]
    return (out, vq, 0.25 * vq, jnp.float32(0.0), use_s[0, 0], top1_s[0, 0])


# speedup vs baseline: 2.0669x; 1.0259x over previous
import functools

import jax
import jax.numpy as jnp
from jax.experimental import pallas as pl
from jax.experimental.pallas import tpu as pltpu
from jax.experimental.pallas import tpu_sc as plsc

_B, _L, _C = 32, 450, 768
_MID = 256
_K = 8912
_KP = 9216
_N = _B * _L
_TN = 1800
_TK = 512
_NT = _N // _TN
_NK = _KP // _TK
_TK0 = 1024
_INV_TEMP = -1.0 / 0.07
_EPS = 1e-12

_INTERPRET = False
_PREC = jax.lax.Precision.DEFAULT


def _up_body(zq_ref, wup_ref, bup_ref, out_ref):
    out_ref[...] = jnp.dot(zq_ref[...], wup_ref[...],
                           preferred_element_type=jnp.float32) + bup_ref[...]


def _flash_body(zn_ref, enT_ref, dmin_ref, idx_ref, se_ref,
                rmin_ref, ridx_ref, rsum_ref):
    k = pl.program_id(1)

    @pl.when(k == 0)
    def _():
        rmin_ref[...] = jnp.full((_TN, 1), 3e38, jnp.float32)
        ridx_ref[...] = jnp.full((_TN, 1), 2**30, jnp.int32)
        rsum_ref[...] = jnp.zeros((_TN, 1), jnp.float32)

    en_k = enT_ref[:, pl.ds(k * _TK, _TK)]
    s = jnp.dot(zn_ref[...], en_k, precision=_PREC,
                preferred_element_type=jnp.float32)
    d = -2.0 * s + 2.0
    lane = jax.lax.broadcasted_iota(jnp.int32, (1, _TK), 1)
    tail_bias = jnp.where((lane + k * _TK) >= _K, jnp.float32(1e30),
                          jnp.float32(0.0))
    d = d + tail_bias
    tmin = jnp.min(d, axis=1, keepdims=True)
    col = jax.lax.broadcasted_iota(jnp.int32, (_TN, _TK), 1) + k * _TK
    tidx = jnp.min(jnp.where(d == tmin, col, jnp.int32(2**30)),
                   axis=1, keepdims=True)
    e = jnp.exp(d * jnp.float32(_INV_TEMP))
    tsum = jnp.sum(e, axis=1, keepdims=True)

    better = tmin < rmin_ref[...]
    ridx_ref[...] = jnp.where(better, tidx, ridx_ref[...])
    rmin_ref[...] = jnp.where(better, tmin, rmin_ref[...])
    rsum_ref[...] = rsum_ref[...] + tsum

    @pl.when(k == _NK - 1)
    def _():
        dmin_ref[...] = rmin_ref[...]
        idx_ref[...] = ridx_ref[...]
        se_ref[...] = rsum_ref[...]


def _scalars_body(dmin_ref, se_ref, idx_ref, vq_ref, use_ref, top1_ref):
    dm = dmin_ref[...]
    se = se_ref[...]
    vq_ref[...] = (jnp.sum(dm) / jnp.float32(_N * _MID)).reshape(1, 1)
    t1 = jnp.exp(dm * jnp.float32(_INV_TEMP)) / se
    top1_ref[...] = (jnp.sum(t1) / jnp.float32(_N)).reshape(1, 1)

    c_chunk, t_chunk = 1024, 1800

    def code_tile(j, count):
        codes = jax.lax.broadcasted_iota(jnp.int32, (1, c_chunk), 1) \
            + j * c_chunk

        def tok_tile(i, acc):
            chunk = idx_ref[pl.ds(i * t_chunk, t_chunk), :]
            eq = (chunk == codes).astype(jnp.float32)
            return jnp.maximum(acc, jnp.max(eq, axis=0, keepdims=True))

        pres = jax.lax.fori_loop(
            0, _N // t_chunk, tok_tile,
            jnp.zeros((1, c_chunk), jnp.float32))
        return count + jnp.sum(pres)

    count = jax.lax.fori_loop(0, _KP // c_chunk, code_tile, jnp.float32(0.0))
    use_ref[...] = (count / jnp.float32(_K)).reshape(1, 1)


_NG = 14464


def _sc_gather(en, idx2):
    mesh = plsc.VectorSubcoreMesh(core_axis_name="core",
                                  subcore_axis_name="subcore")
    win = 128

    @functools.partial(
        pl.kernel,
        out_type=jax.ShapeDtypeStruct((_NG, _MID), jnp.float32),
        mesh=mesh)
    def gk(en_hbm, i_hbm, o_hbm):
        def body(i_vmem, o_vmem):
            pltpu.sync_copy(en_hbm.at[i_vmem.at[0]], o_vmem)

        pltpu.emit_pipeline(
            body,
            grid=(_NG // win,),
            in_specs=[pl.BlockSpec((1, win), lambda i: (0, i))],
            out_specs=[pl.BlockSpec((win, _MID), lambda i: (i, 0))],
            core_axis_name=("core", "subcore"),
            dimension_semantics=(pltpu.PARALLEL,),
        )(i_hbm, o_hbm)

    return gk(en, idx2)


def kernel(x, calib, emb, W_down, b_down, W_up, b_up):
    del calib
    z = x @ W_down + b_down
    zf = z.reshape(_N, _MID)
    zn = zf / jnp.maximum(
        jnp.sqrt(jnp.sum(zf * zf, axis=-1, keepdims=True)), _EPS)
    en = emb / jnp.maximum(
        jnp.sqrt(jnp.sum(emb * emb, axis=-1, keepdims=True)), _EPS)
    enP = jnp.pad(en, ((0, _KP - _K), (0, 0)))
    enT = enP.T

    dmin, idx, se = pl.pallas_call(
        _flash_body,
        grid=(_NT, _NK),
        in_specs=[
            pl.BlockSpec((_TN, _MID), lambda t, k: (t, 0)),
            pl.BlockSpec((_MID, _KP), lambda t, k: (0, 0)),
        ],
        out_specs=[
            pl.BlockSpec((_TN, 1), lambda t, k: (t, 0)),
            pl.BlockSpec((_TN, 1), lambda t, k: (t, 0)),
            pl.BlockSpec((_TN, 1), lambda t, k: (t, 0)),
        ],
        out_shape=[
            jax.ShapeDtypeStruct((_N, 1), jnp.float32),
            jax.ShapeDtypeStruct((_N, 1), jnp.int32),
            jax.ShapeDtypeStruct((_N, 1), jnp.float32),
        ],
        scratch_shapes=[
            pltpu.VMEM((_TN, 1), jnp.float32),
            pltpu.VMEM((_TN, 1), jnp.int32),
            pltpu.VMEM((_TN, 1), jnp.float32),
        ],
        interpret=_INTERPRET,
    )(zn, enT)

    vq_s, use_s, top1_s = pl.pallas_call(
        _scalars_body,
        in_specs=[
            pl.BlockSpec((_N, 1), lambda: (0, 0)),
            pl.BlockSpec((_N, 1), lambda: (0, 0)),
            pl.BlockSpec((_N, 1), lambda: (0, 0)),
        ],
        out_specs=[
            pl.BlockSpec((1, 1), lambda: (0, 0)),
            pl.BlockSpec((1, 1), lambda: (0, 0)),
            pl.BlockSpec((1, 1), lambda: (0, 0)),
        ],
        out_shape=[
            jax.ShapeDtypeStruct((1, 1), jnp.float32),
            jax.ShapeDtypeStruct((1, 1), jnp.float32),
            jax.ShapeDtypeStruct((1, 1), jnp.float32),
        ],
        interpret=_INTERPRET,
    )(dmin, se, idx)

    if _INTERPRET:
        zq = jnp.take(enP, idx[:, 0], axis=0)
    else:
        idxp = jnp.pad(idx.reshape(1, _N), ((0, 0), (0, _NG - _N)))
        zq = _sc_gather(enP, idxp)[:_N]

    out2d = pl.pallas_call(
        _up_body,
        grid=(_NT,),
        in_specs=[
            pl.BlockSpec((_TN, _MID), lambda t: (t, 0)),
            pl.BlockSpec((_MID, _C), lambda t: (0, 0)),
            pl.BlockSpec((1, _C), lambda t: (0, 0)),
        ],
        out_specs=pl.BlockSpec((_TN, _C), lambda t: (t, 0)),
        out_shape=jax.ShapeDtypeStruct((_N, _C), jnp.float32),
        interpret=_INTERPRET,
    )(zq, W_up, b_up.reshape(1, _C))

    out = out2d.reshape(_B, _L, _C)
    vq = vq_s[0, 0]
    return (out, vq, 0.25 * vq, jnp.float32(0.0), use_s[0, 0], top1_s[0, 0])


# speedup vs baseline: 2.3056x; 1.1155x over previous
import functools

import jax
import jax.numpy as jnp
from jax.experimental import pallas as pl
from jax.experimental.pallas import tpu as pltpu
from jax.experimental.pallas import tpu_sc as plsc

_B, _L, _C = 32, 450, 768
_MID = 256
_K = 8912
_KP = 9216
_N = _B * _L
_TN = 1800
_TK = 1024
_NT = _N // _TN
_NK = _KP // _TK
_TK0 = 1024
_INV_TEMP = -1.0 / 0.07
_EPS = 1e-12

_INTERPRET = False
_PREC = jax.lax.Precision.DEFAULT


def _up_body(zq_ref, wup_ref, bup_ref, out_ref):
    out_ref[...] = jnp.dot(zq_ref[...], wup_ref[...],
                           preferred_element_type=jnp.float32) + bup_ref[...]


def _flash_body(zn_ref, enT_ref, dmin_ref, idx_ref, se_ref,
                rmin_ref, ridx_ref, rsum_ref):
    k = pl.program_id(1)

    @pl.when(k == 0)
    def _():
        rmin_ref[...] = jnp.full((_TN, 1), 3e38, jnp.float32)
        ridx_ref[...] = jnp.full((_TN, 1), 2**30, jnp.int32)
        rsum_ref[...] = jnp.zeros((_TN, 1), jnp.float32)

    en_k = enT_ref[:, pl.ds(k * _TK, _TK)]
    s = jnp.dot(zn_ref[...], en_k, precision=_PREC,
                preferred_element_type=jnp.float32)
    d = -2.0 * s + 2.0
    tmin = jnp.min(d, axis=1, keepdims=True)
    col = jax.lax.broadcasted_iota(jnp.int32, (_TN, _TK), 1) + k * _TK
    tidx = jnp.min(jnp.where(d == tmin, col, jnp.int32(2**30)),
                   axis=1, keepdims=True)
    e = jnp.exp(d * jnp.float32(_INV_TEMP))
    tsum = jnp.sum(e, axis=1, keepdims=True)

    better = tmin < rmin_ref[...]
    ridx_ref[...] = jnp.where(better, tidx, ridx_ref[...])
    rmin_ref[...] = jnp.where(better, tmin, rmin_ref[...])
    rsum_ref[...] = rsum_ref[...] + tsum

    @pl.when(k == _NK - 1)
    def _():
        dmin_ref[...] = rmin_ref[...]
        idx_ref[...] = ridx_ref[...]
        se_ref[...] = rsum_ref[...]


def _scalars_body(dmin_ref, se_ref, vq_ref, top1_ref):
    dm = dmin_ref[...]
    padc = jnp.float32(_KP - _K) * jnp.exp(
        jnp.float32(2.0) * jnp.float32(_INV_TEMP))
    se = se_ref[...] - padc
    vq_ref[...] = (jnp.sum(dm) / jnp.float32(_N * _MID)).reshape(1, 1)
    t1 = jnp.exp(dm * jnp.float32(_INV_TEMP)) / se
    top1_ref[...] = (jnp.sum(t1) / jnp.float32(_N)).reshape(1, 1)


_CC = 1024


def _presence_body(idx_ref, pc_ref):
    j = pl.program_id(0)
    codes = jax.lax.broadcasted_iota(jnp.int32, (1, _CC), 1) + j * _CC
    t_chunk = 1800

    def tok_tile(i, acc):
        chunk = idx_ref[pl.ds(i * t_chunk, t_chunk), :]
        eq = (chunk == codes).astype(jnp.float32)
        return jnp.maximum(acc, jnp.max(eq, axis=0, keepdims=True))

    pres = jax.lax.fori_loop(
        0, _N // t_chunk, tok_tile, jnp.zeros((1, _CC), jnp.float32))
    pc_ref[...] = jnp.sum(pres).reshape(1, 1, 1)


_NG = 14464


def _sc_gather(en, idx2):
    mesh = plsc.VectorSubcoreMesh(core_axis_name="core",
                                  subcore_axis_name="subcore")
    win = 128

    @functools.partial(
        pl.kernel,
        out_type=jax.ShapeDtypeStruct((_NG, _MID), jnp.float32),
        mesh=mesh)
    def gk(en_hbm, i_hbm, o_hbm):
        def body(i_vmem, o_vmem):
            pltpu.sync_copy(en_hbm.at[i_vmem.at[0]], o_vmem)

        pltpu.emit_pipeline(
            body,
            grid=(_NG // win,),
            in_specs=[pl.BlockSpec((1, win), lambda i: (0, i))],
            out_specs=[pl.BlockSpec((win, _MID), lambda i: (i, 0))],
            core_axis_name=("core", "subcore"),
            dimension_semantics=(pltpu.PARALLEL,),
        )(i_hbm, o_hbm)

    return gk(en, idx2)


def kernel(x, calib, emb, W_down, b_down, W_up, b_up):
    del calib
    z = x @ W_down + b_down
    zf = z.reshape(_N, _MID)
    zn = zf / jnp.maximum(
        jnp.sqrt(jnp.sum(zf * zf, axis=-1, keepdims=True)), _EPS)
    en = emb / jnp.maximum(
        jnp.sqrt(jnp.sum(emb * emb, axis=-1, keepdims=True)), _EPS)
    enP = jnp.pad(en, ((0, _KP - _K), (0, 0)))
    enT = enP.T

    dmin, idx, se = pl.pallas_call(
        _flash_body,
        grid=(_NT, _NK),
        in_specs=[
            pl.BlockSpec((_TN, _MID), lambda t, k: (t, 0)),
            pl.BlockSpec((_MID, _KP), lambda t, k: (0, 0)),
        ],
        out_specs=[
            pl.BlockSpec((_TN, 1), lambda t, k: (t, 0)),
            pl.BlockSpec((_TN, 1), lambda t, k: (t, 0)),
            pl.BlockSpec((_TN, 1), lambda t, k: (t, 0)),
        ],
        out_shape=[
            jax.ShapeDtypeStruct((_N, 1), jnp.float32),
            jax.ShapeDtypeStruct((_N, 1), jnp.int32),
            jax.ShapeDtypeStruct((_N, 1), jnp.float32),
        ],
        scratch_shapes=[
            pltpu.VMEM((_TN, 1), jnp.float32),
            pltpu.VMEM((_TN, 1), jnp.int32),
            pltpu.VMEM((_TN, 1), jnp.float32),
        ],
        compiler_params=pltpu.CompilerParams(
            dimension_semantics=("parallel", "arbitrary")),
        interpret=_INTERPRET,
    )(zn, enT)

    vq_s, top1_s = pl.pallas_call(
        _scalars_body,
        in_specs=[
            pl.BlockSpec((_N, 1), lambda: (0, 0)),
            pl.BlockSpec((_N, 1), lambda: (0, 0)),
        ],
        out_specs=[
            pl.BlockSpec((1, 1), lambda: (0, 0)),
            pl.BlockSpec((1, 1), lambda: (0, 0)),
        ],
        out_shape=[
            jax.ShapeDtypeStruct((1, 1), jnp.float32),
            jax.ShapeDtypeStruct((1, 1), jnp.float32),
        ],
        interpret=_INTERPRET,
    )(dmin, se)

    pc = pl.pallas_call(
        _presence_body,
        grid=(_KP // _CC,),
        in_specs=[pl.BlockSpec((_N, 1), lambda j: (0, 0))],
        out_specs=pl.BlockSpec((1, 1, 1), lambda j: (j, 0, 0)),
        out_shape=jax.ShapeDtypeStruct((_KP // _CC, 1, 1), jnp.float32),
        compiler_params=pltpu.CompilerParams(
            dimension_semantics=("parallel",)),
        interpret=_INTERPRET,
    )(idx)
    use_s = jnp.sum(pc) / jnp.float32(_K)

    if _INTERPRET:
        zq = jnp.take(enP, idx[:, 0], axis=0)
    else:
        idxp = jnp.pad(idx.reshape(1, _N), ((0, 0), (0, _NG - _N)))
        zq = _sc_gather(enP, idxp)[:_N]

    out2d = pl.pallas_call(
        _up_body,
        grid=(_NT,),
        in_specs=[
            pl.BlockSpec((_TN, _MID), lambda t: (t, 0)),
            pl.BlockSpec((_MID, _C), lambda t: (0, 0)),
            pl.BlockSpec((1, _C), lambda t: (0, 0)),
        ],
        out_specs=pl.BlockSpec((_TN, _C), lambda t: (t, 0)),
        out_shape=jax.ShapeDtypeStruct((_N, _C), jnp.float32),
        compiler_params=pltpu.CompilerParams(
            dimension_semantics=("parallel",)),
        interpret=_INTERPRET,
    )(zq, W_up, b_up.reshape(1, _C))

    out = out2d.reshape(_B, _L, _C)
    vq = vq_s[0, 0]
    return (out, vq, 0.25 * vq, jnp.float32(0.0), use_s, top1_s[0, 0])
